# Initial kernel scaffold; baseline (speedup 1.0000x reference)
import jax
import jax.numpy as jnp
from jax.experimental import pallas as pl


def kernel(table, idxes):
    raise NotImplementedError("write your pallas kernel here")


# speedup vs baseline: 2.2524x; 2.2524x over previous
import functools

import jax
import jax.numpy as jnp
from jax import lax
from jax.experimental import pallas as pl
from jax.experimental.pallas import tpu as pltpu
from jax.experimental.pallas import tpu_sc as plsc

_DIM = 300
_CHUNK = 128


@functools.partial(jax.jit, static_argnames=("n_rows",))
def _gather(table, idx_flat, n_rows):
    info = plsc.get_sparse_core_info()
    nc, ns = info.num_cores, info.num_subcores
    nw = nc * ns
    chunks_per_w = n_rows // (_CHUNK * nw)
    mesh = plsc.VectorSubcoreMesh(core_axis_name="c", subcore_axis_name="s")

    @functools.partial(
        pl.kernel,
        mesh=mesh,
        out_type=jax.ShapeDtypeStruct((n_rows, _DIM), jnp.float32),
        scratch_types=[
            pltpu.VMEM((_CHUNK,), jnp.int32),
            pltpu.VMEM((_CHUNK, _DIM), jnp.float32),
            pltpu.SemaphoreType.DMA,
        ],
    )
    def k(table_hbm, idx_hbm, out_hbm, idx_v, rows_v, sem):
        wid = lax.axis_index("s") * nc + lax.axis_index("c")
        c0 = wid * chunks_per_w

        def body(j, carry):
            base = (c0 + j) * _CHUNK
            pltpu.sync_copy(idx_hbm.at[pl.ds(base, _CHUNK)], idx_v)

            def group(g, carry2):
                vec = idx_v[pl.ds(g * 16, 16)]
                for l in range(16):
                    r = vec[l]
                    pltpu.async_copy(
                        table_hbm.at[pl.ds(r, 1)],
                        rows_v.at[pl.ds(g * 16 + l, 1)],
                        sem,
                    )
                return carry2

            lax.fori_loop(0, _CHUNK // 16, group, 0)
            pltpu.make_async_copy(
                table_hbm.at[pl.ds(0, _CHUNK)], rows_v, sem
            ).wait()
            pltpu.sync_copy(rows_v, out_hbm.at[pl.ds(base, _CHUNK)])
            return carry

        lax.fori_loop(0, chunks_per_w, body, 0)

    return k(table, idx_flat)


def kernel(table, idxes):
    b0, b1 = idxes.shape
    n_rows = b0 * b1
    idx_flat = idxes.reshape(n_rows).astype(jnp.int32)
    out = _gather(table, idx_flat, n_rows)
    return out.reshape(b0, b1, _DIM)


# speedup vs baseline: 2.4150x; 1.0722x over previous
import functools

import jax
import jax.numpy as jnp
from jax import lax
from jax.experimental import pallas as pl
from jax.experimental.pallas import tpu as pltpu
from jax.experimental.pallas import tpu_sc as plsc

_DIM = 300
_CHUNK = 128
_SUP = 20


@functools.partial(jax.jit, static_argnames=("n_rows",))
def _gather(table, idx_flat, n_rows):
    info = plsc.get_sparse_core_info()
    nc, ns = info.num_cores, info.num_subcores
    nw = nc * ns
    chunks_per_w = n_rows // (_CHUNK * nw)
    n_sup = chunks_per_w // _SUP
    mesh = plsc.VectorSubcoreMesh(core_axis_name="c", subcore_axis_name="s")

    @functools.partial(
        pl.kernel,
        mesh=mesh,
        out_type=jax.ShapeDtypeStruct((n_rows, _DIM), jnp.float32),
        scratch_types=[
            pltpu.VMEM((_SUP * _CHUNK,), jnp.int32),
            pltpu.VMEM((_CHUNK, _DIM), jnp.float32),
            pltpu.VMEM((_CHUNK, _DIM), jnp.float32),
            pltpu.SemaphoreType.DMA,
            pltpu.SemaphoreType.DMA,
            pltpu.SemaphoreType.DMA,
            pltpu.SemaphoreType.DMA,
        ],
    )
    def k(table_hbm, idx_hbm, out_hbm, idx_v, rows0, rows1, g0, g1, w0, w1):
        wid = lax.axis_index("s") * nc + lax.axis_index("c")
        c0 = wid * chunks_per_w
        rows = (rows0, rows1)
        sem_g = (g0, g1)
        sem_w = (w0, w1)

        def issue_gathers(j_local, b):
            def group(g, carry):
                vec = idx_v[pl.ds(j_local * _CHUNK + g * 16, 16)]
                for l in range(16):
                    pltpu.async_copy(
                        table_hbm.at[pl.ds(vec[l], 1)],
                        rows[b].at[pl.ds(g * 16 + l, 1)],
                        sem_g[b],
                    )
                return carry

            lax.fori_loop(0, _CHUNK // 16, group, 0)

        def drain_gathers(b):
            pltpu.make_async_copy(
                table_hbm.at[pl.ds(0, _CHUNK)], rows[b], sem_g[b]
            ).wait()

        def write_out(sup, j_local, b):
            base = (c0 + sup * _SUP + j_local) * _CHUNK
            pltpu.async_copy(rows[b], out_hbm.at[pl.ds(base, _CHUNK)], sem_w[b])

        def wait_write(b):
            pltpu.make_async_copy(
                rows[b], out_hbm.at[pl.ds(0, _CHUNK)], sem_w[b]
            ).wait()

        def super_body(sup, carry):
            pltpu.sync_copy(
                idx_hbm.at[pl.ds((c0 + sup * _SUP) * _CHUNK, _SUP * _CHUNK)],
                idx_v,
            )
            pl.when(sup > 0)(lambda: wait_write(0))
            issue_gathers(0, 0)

            def pair(p, carry2):
                pl.when(sup + p > 0)(lambda: wait_write(1))
                issue_gathers(2 * p + 1, 1)
                drain_gathers(0)
                write_out(sup, 2 * p, 0)
                wait_write(0)
                issue_gathers(2 * p + 2, 0)
                drain_gathers(1)
                write_out(sup, 2 * p + 1, 1)
                return carry2

            lax.fori_loop(0, (_SUP - 2) // 2, pair, 0)
            wait_write(1)
            issue_gathers(_SUP - 1, 1)
            drain_gathers(0)
            write_out(sup, _SUP - 2, 0)
            drain_gathers(1)
            write_out(sup, _SUP - 1, 1)
            return carry

        lax.fori_loop(0, n_sup, super_body, 0)
        wait_write(0)
        wait_write(1)

    return k(table, idx_flat)


def kernel(table, idxes):
    b0, b1 = idxes.shape
    n_rows = b0 * b1
    idx_flat = idxes.reshape(n_rows).astype(jnp.int32)
    out = _gather(table, idx_flat, n_rows)
    return out.reshape(b0, b1, _DIM)
